# baseline (speedup 1.0000x reference)
import functools

import jax
import jax.numpy as jnp
from jax import lax
from jax.experimental import pallas as pl
from jax.experimental.pallas import tpu as pltpu
from jax.experimental.pallas import tpu_sc as plsc

MAXV = 100000
NCH = 26
DIM = 16
LANES = 16
CHUNK = 128


def _bow_kernel(B, nb):
    nsub = nb // CHUNK
    mesh = plsc.VectorSubcoreMesh(core_axis_name="c", subcore_axis_name="s")

    @functools.partial(
        pl.kernel,
        mesh=mesh,
        out_type=jax.ShapeDtypeStruct((B, NCH * DIM), jnp.float32),
        compiler_params=pltpu.CompilerParams(use_tc_tiling_on_sc=False),
        scratch_types=[
            pltpu.VMEM((nsub, CHUNK), jnp.int32),
            pltpu.VMEM((nb, DIM), jnp.float32),
            pltpu.SemaphoreType.DMA,
        ],
    )
    def k(idx_hbm, table_hbm, out_hbm, idx_v, rows_v, sem):
        wid = lax.axis_index("s") * 2 + lax.axis_index("c")
        base = wid * nb

        def chan(c, _):
            pltpu.sync_copy(idx_hbm.at[c, pl.ds(wid * nsub, nsub)], idx_v)
            off = c * MAXV

            def addb(j, _):
                row = j // (CHUNK // LANES)
                col = (j % (CHUNK // LANES)) * LANES
                sl = pl.ds(col, LANES)
                idx_v[row, sl] = idx_v[row, sl] + off
                return 0

            lax.fori_loop(0, nb // LANES, addb, 0)

            for j in range(nsub):
                pltpu.async_copy(
                    table_hbm.at[idx_v.at[j]],
                    rows_v.at[pl.ds(j * CHUNK, CHUNK)],
                    sem,
                )
            for j in range(nsub):
                pltpu.make_async_copy(
                    table_hbm.at[idx_v.at[j]],
                    rows_v.at[pl.ds(j * CHUNK, CHUNK)],
                    sem,
                ).wait()
            pltpu.sync_copy(
                rows_v, out_hbm.at[pl.ds(base, nb), pl.ds(c * DIM, DIM)]
            )
            return 0

        lax.fori_loop(0, NCH, chan, 0)

    return k


def kernel(inputs, table):
    orig_shape = inputs.shape
    flat = inputs.reshape(-1, orig_shape[-1])
    B = flat.shape[0]
    nw = 32
    nb = B // nw
    idx3 = flat.astype(jnp.int32).T.reshape(NCH, B // CHUNK, CHUNK)
    out = _bow_kernel(B, nb)(idx3, table)
    return out.reshape(orig_shape[:-1] + (NCH * DIM,))


# speedup vs baseline: 1.0302x; 1.0302x over previous
import functools

import jax
import jax.numpy as jnp
from jax import lax
from jax.experimental import pallas as pl
from jax.experimental.pallas import tpu as pltpu
from jax.experimental.pallas import tpu_sc as plsc

MAXV = 100000
NCH = 26
DIM = 16
LANES = 16
CHUNK = 128
NW = 32
NBUF = 4
AHEAD = 2


def _bow_kernel(B, nb):
    nsub = nb // CHUNK

    mesh = plsc.VectorSubcoreMesh(core_axis_name="c", subcore_axis_name="s")

    @functools.partial(
        pl.kernel,
        mesh=mesh,
        out_type=jax.ShapeDtypeStruct((B, NCH * DIM), jnp.float32),
        compiler_params=pltpu.CompilerParams(use_tc_tiling_on_sc=False),
        scratch_types=[
            pltpu.VMEM((NCH, nsub, CHUNK), jnp.int32),
            pltpu.VMEM((NBUF, nb, DIM), jnp.float32),
            pltpu.SemaphoreType.DMA,
            pltpu.SemaphoreType.DMA,
        ],
    )
    def k(idx_hbm, table_hbm, out_hbm, idx_v, rows_v, gsem, wsem):
        wid = lax.axis_index("s") * 2 + lax.axis_index("c")
        base = wid * nb

        pltpu.sync_copy(idx_hbm.at[wid], idx_v)

        def add_offsets(c):
            off = c * MAXV
            for jj in range(nb // LANES):
                r, col = jj // (CHUNK // LANES), (jj % (CHUNK // LANES)) * LANES
                sl = pl.ds(col, LANES)
                idx_v[c, r, sl] = idx_v[c, r, sl] + off

        def fire_gathers(c, buf):
            for j in range(nsub):
                pltpu.async_copy(
                    table_hbm.at[idx_v.at[c, j]],
                    rows_v.at[buf, pl.ds(j * CHUNK, CHUNK)],
                    gsem,
                )

        def wait_gathers(c, buf):
            for j in range(nsub):
                pltpu.make_async_copy(
                    table_hbm.at[idx_v.at[c, j]],
                    rows_v.at[buf, pl.ds(j * CHUNK, CHUNK)],
                    gsem,
                ).wait()

        def out_slice(c):
            return out_hbm.at[pl.ds(base, nb), pl.ds(c * DIM, DIM)]

        for p in range(AHEAD):
            add_offsets(p)
            fire_gathers(p, p % NBUF)

        def body(c, _):
            buf = lax.rem(c, NBUF)

            @pl.when(c >= AHEAD)
            def _():
                pltpu.make_async_copy(
                    rows_v.at[lax.rem(c - AHEAD, NBUF)],
                    out_slice(c - AHEAD),
                    wsem,
                ).wait()

            @pl.when(c + AHEAD < NCH)
            def _():
                add_offsets(c + AHEAD)
                fire_gathers(c + AHEAD, lax.rem(c + AHEAD, NBUF))

            wait_gathers(c, buf)
            pltpu.async_copy(rows_v.at[buf], out_slice(c), wsem)
            return 0

        lax.fori_loop(0, NCH, body, 0)

        for c in range(NCH - AHEAD, NCH):
            pltpu.make_async_copy(
                rows_v.at[c % NBUF], out_slice(c), wsem
            ).wait()

    return k


def kernel(inputs, table):
    orig_shape = inputs.shape
    flat = inputs.reshape(-1, orig_shape[-1])
    B = flat.shape[0]
    nb = B // NW
    nsub = nb // CHUNK
    idx4 = (
        flat.astype(jnp.int32)
        .reshape(NW, nb, NCH)
        .transpose(0, 2, 1)
        .reshape(NW, NCH, nsub, CHUNK)
    )
    out = _bow_kernel(B, nb)(idx4, table)
    return out.reshape(orig_shape[:-1] + (NCH * DIM,))


# speedup vs baseline: 1.0342x; 1.0038x over previous
import functools

import jax
import jax.numpy as jnp
from jax import lax
from jax.experimental import pallas as pl
from jax.experimental.pallas import tpu as pltpu
from jax.experimental.pallas import tpu_sc as plsc

MAXV = 100000
NCH = 26
DIM = 16
LANES = 16
CHUNK = 128
NW = 32
PER = 13
SPC = 13
NBUF = 4
AHEAD = 2


def _bow_kernel(total, npw):
    nrows = npw // CHUNK
    nchunk = nrows // SPC
    crows = SPC * CHUNK

    mesh = plsc.VectorSubcoreMesh(core_axis_name="c", subcore_axis_name="s")

    @functools.partial(
        pl.kernel,
        mesh=mesh,
        out_type=jax.ShapeDtypeStruct((total, DIM), jnp.float32),
        compiler_params=pltpu.CompilerParams(use_tc_tiling_on_sc=False),
        scratch_types=[
            pltpu.VMEM((nrows, CHUNK), jnp.int32),
            pltpu.VMEM((NBUF, crows, DIM), jnp.float32),
            pltpu.SemaphoreType.DMA,
            pltpu.SemaphoreType.DMA,
        ],
    )
    def k(idx_hbm, table_hbm, out_hbm, idx_v, rows_v, gsem, wsem):
        wid = lax.axis_index("s") * 2 + lax.axis_index("c")
        base = wid * npw

        pltpu.sync_copy(idx_hbm.at[wid], idx_v)

        lane = lax.iota(jnp.int32, LANES)
        pat = [
            ((lane + j * LANES) % NCH) * MAXV for j in range(PER)
        ]

        def add_offsets(c):
            def grp(g, _):
                q0 = c * crows + g * (PER * LANES)
                for j in range(PER):
                    q = q0 + j * LANES
                    r = lax.shift_right_logical(q, 7)
                    col = lax.bitwise_and(q, 127)
                    sl = pl.ds(col, LANES)
                    idx_v[r, sl] = idx_v[r, sl] + pat[j]
                return 0

            lax.fori_loop(0, crows // (PER * LANES), grp, 0)

        def fire_gathers(c, buf):
            for j in range(SPC):
                pltpu.async_copy(
                    table_hbm.at[idx_v.at[c * SPC + j]],
                    rows_v.at[buf, pl.ds(j * CHUNK, CHUNK)],
                    gsem,
                )

        def wait_gathers(c, buf):
            for j in range(SPC):
                pltpu.make_async_copy(
                    table_hbm.at[idx_v.at[c * SPC + j]],
                    rows_v.at[buf, pl.ds(j * CHUNK, CHUNK)],
                    gsem,
                ).wait()

        def out_slice(c):
            return out_hbm.at[pl.ds(base + c * crows, crows)]

        for p in range(AHEAD):
            add_offsets(p)
            fire_gathers(p, p % NBUF)

        def body(c, _):
            buf = lax.rem(c, NBUF)

            @pl.when(c >= AHEAD)
            def _():
                pltpu.make_async_copy(
                    rows_v.at[lax.rem(c - AHEAD, NBUF)],
                    out_slice(c - AHEAD),
                    wsem,
                ).wait()

            @pl.when(c + AHEAD < nchunk)
            def _():
                add_offsets(c + AHEAD)
                fire_gathers(c + AHEAD, lax.rem(c + AHEAD, NBUF))

            wait_gathers(c, buf)
            pltpu.async_copy(rows_v.at[buf], out_slice(c), wsem)
            return 0

        lax.fori_loop(0, nchunk, body, 0)

        for c in range(nchunk - AHEAD, nchunk):
            pltpu.make_async_copy(
                rows_v.at[c % NBUF], out_slice(c), wsem
            ).wait()

    return k


def kernel(inputs, table):
    orig_shape = inputs.shape
    flat = inputs.reshape(-1, orig_shape[-1])
    B = flat.shape[0]
    total = B * NCH
    npw = total // NW
    idx3 = flat.astype(jnp.int32).reshape(NW, npw // CHUNK, CHUNK)
    out = _bow_kernel(total, npw)(idx3, table)
    return out.reshape(orig_shape[:-1] + (NCH * DIM,))
